# Initial kernel scaffold; baseline (speedup 1.0000x reference)
import jax
import jax.numpy as jnp
from jax.experimental import pallas as pl


def kernel(language, video, audio, missing_index, W_language, b_language, W_video, b_video, W_audio, b_audio, W1, b1, W2, b2):
    raise NotImplementedError("write your pallas kernel here")


# speedup vs baseline: 15.1346x; 15.1346x over previous
import jax
import jax.numpy as jnp
from jax.experimental import pallas as pl
from jax.experimental.pallas import tpu as pltpu

B = 1024
D = 768
FD = 768
BANK = 1000
NEG = -1e30


def _dot(a, b, dims):
    return jax.lax.dot_general(a, b, (dims, ((), ())),
                               precision=jax.lax.Precision.DEFAULT,
                               preferred_element_type=jnp.float32)


def _step_kernel(mi_ref, x_ref, w_ref, b_ref, w1_ref, b1_ref, w2_ref, b2_ref,
                 out_ref, acc_ref):
    m = pl.program_id(0)
    t = m + 1

    mi = mi_ref[...]
    missing_row = (mi == t)
    avail_row = jnp.where(missing_row, 0.0, 1.0)

    proj = _dot(x_ref[0], w_ref[0], (((1,), (0,)))) + b_ref[0]

    ii = jax.lax.broadcasted_iota(jnp.int32, (B, B), 0)
    jj = jax.lax.broadcasted_iota(jnp.int32, (B, B), 1)
    triu = jnp.where(ii <= jj, 1.0, 0.0)
    csum = _dot(avail_row, triu, (((1,), (0,))))

    A = jnp.sum(avail_row, keepdims=True)
    off = jnp.maximum(A - BANK, 0.0)
    Ae = jnp.minimum(A, float(BANK))

    slot = csum - 1.0 - off

    pidx = ii.astype(jnp.float32)
    P = jnp.where((pidx == slot) & (avail_row > 0.0), 1.0, 0.0)
    Fb = _dot(P, proj, (((1,), (0,))))

    nrm2 = jnp.sum(Fb * Fb, axis=1, keepdims=True)
    nrm = jnp.sqrt(nrm2)
    dotm = _dot(Fb, Fb, (((1,), (1,))))
    nprod = nrm * jnp.transpose(nrm)
    sim = dotm / jnp.maximum(nprod, 1e-8)

    jrow = jax.lax.broadcasted_iota(jnp.int32, (1, B), 1).astype(jnp.float32)
    colvalid = (avail_row > 0.0) & (jrow < Ae) & (jrow < float(BANK))
    L = jnp.sum(jnp.where(colvalid, 1.0, 0.0), keepdims=True)
    v = jnp.where(colvalid, sim, NEG)

    jf = jj
    tvs, tis = [], []
    for _ in range(3):
        mval = jnp.max(v, axis=1, keepdims=True)
        hit = (v == mval)
        midx = jnp.min(jnp.where(hit, jf, B), axis=1, keepdims=True)
        tvs.append(mval)
        tis.append(midx)
        v = jnp.where(jf == midx, NEG, v)

    e1 = jnp.ones_like(tvs[0])
    e2 = jnp.exp(tvs[1] - tvs[0])
    e3 = jnp.exp(tvs[2] - tvs[0])
    den = e1 + e2 + e3
    w1 = e1 / den
    w2 = e2 / den
    w3 = e3 / den

    S = (jnp.where(jf == tis[0], w1, 0.0)
         + jnp.where(jf == tis[1], w2, 0.0)
         + jnp.where(jf == tis[2], w3, 0.0))
    knn = _dot(S, Fb, (((1,), (0,))))

    icol = jax.lax.broadcasted_iota(jnp.int32, (B, 1), 0).astype(jnp.float32)
    use = (icol < Ae) & (L > 0.0)
    missing_col = jnp.transpose(missing_row)
    knn = jnp.where(use, knn, 0.0)
    filled = jnp.where(missing_col, knn, proj)

    contrib = _dot(filled, w1_ref[...], (((1,), (0,))))

    @pl.when(m == 0)
    def _():
        acc_ref[...] = contrib

    @pl.when(m > 0)
    def _():
        acc_ref[...] = acc_ref[...] + contrib

    @pl.when(m == 2)
    def _():
        h = jnp.maximum(acc_ref[...] + b1_ref[...], 0.0)
        out_ref[...] = _dot(h, w2_ref[...], (((1,), (0,)))) + b2_ref[...]


def kernel(language, video, audio, missing_index, W_language, b_language,
           W_video, b_video, W_audio, b_audio, W1, b1, W2, b2):
    Xs = jnp.stack([language, video, audio])
    Ws = jnp.stack([W_language, W_video, W_audio])
    bs = jnp.stack([b_language, b_video, b_audio])[:, None, :]
    mi = missing_index.astype(jnp.int32).reshape(1, B)

    grid = (3,)
    out = pl.pallas_call(
        _step_kernel,
        grid=grid,
        in_specs=[
            pl.BlockSpec((1, B), lambda m: (0, 0)),
            pl.BlockSpec((1, B, D), lambda m: (m, 0, 0)),
            pl.BlockSpec((1, D, FD), lambda m: (m, 0, 0)),
            pl.BlockSpec((1, 1, FD), lambda m: (m, 0, 0)),
            pl.BlockSpec((FD, FD), lambda m: (m, 0)),
            pl.BlockSpec((1, FD), lambda m: (0, 0)),
            pl.BlockSpec((FD, 1), lambda m: (0, 0)),
            pl.BlockSpec((1, 1), lambda m: (0, 0)),
        ],
        out_specs=pl.BlockSpec((B, 1), lambda m: (0, 0)),
        out_shape=jax.ShapeDtypeStruct((B, 1), jnp.float32),
        scratch_shapes=[pltpu.VMEM((B, FD), jnp.float32)],
    )(mi, Xs, Ws, bs, W1, b1.reshape(1, FD), W2, b2.reshape(1, 1))
    return out
